# baseline (speedup 1.0000x reference)
import functools

import jax
import jax.numpy as jnp
from jax import lax
from jax.experimental import pallas as pl
from jax.experimental.pallas import tpu as pltpu
from jax.experimental.pallas import tpu_sc as plsc

_NC = 2
_NS = 16
_NW = _NC * _NS
_R = 128
_NB = 3
_L = 16
_B_SC = 73728
_TC_BM = 4096


@functools.lru_cache(maxsize=None)
def _build_sc_call(B: int, Bs: int, P: int, F: int):
    rows_w = Bs // _NW
    n_chunks = rows_w // _R
    assert rows_w % _R == 0 and Bs % _NW == 0 and F % _L == 0 and P % _L == 0

    mesh = plsc.VectorSubcoreMesh(core_axis_name="c", subcore_axis_name="s")

    scratch = (
        [pltpu.VMEM((_R, F), jnp.float32) for _ in range(_NB)]
        + [pltpu.VMEM((_R, P), jnp.float32) for _ in range(_NB)]
        + [pltpu.VMEM((F,), jnp.int32),
           pltpu.VMEM((P,), jnp.float32)]
        + [pltpu.SemaphoreType.DMA for _ in range(2 * _NB + 1)]
    )

    @functools.partial(
        pl.kernel,
        out_type=jax.ShapeDtypeStruct((Bs, P), jnp.float32),
        mesh=mesh,
        compiler_params=pltpu.CompilerParams(
            use_tc_tiling_on_sc=True, needs_layout_passes=False),
        scratch_types=scratch,
    )
    def kfn(params_hbm, fi_hbm, dflt_hbm, out_hbm, *refs):
        ins = refs[:_NB]
        obs = refs[_NB:2 * _NB]
        fi_v, dflt_v = refs[2 * _NB:2 * _NB + 2]
        sin = refs[2 * _NB + 2:3 * _NB + 2]
        sout = refs[3 * _NB + 2:4 * _NB + 2]
        sx = refs[4 * _NB + 2]

        wid = lax.axis_index("s") * _NC + lax.axis_index("c")
        base_row = wid * rows_w

        pltpu.async_copy(fi_hbm, fi_v, sx).wait()
        pltpu.async_copy(dflt_hbm, dflt_v, sx).wait()

        fi = [fi_v[pl.ds(k * _L, _L)] for k in range(F // _L)]
        dv = [dflt_v[pl.ds(j * _L, _L)] for j in range(P // _L)]

        zero16 = jnp.zeros((_L,), jnp.int32)

        def init(ob):
            def body(r, carry):
                for j in range(P // _L):
                    ob[r, pl.ds(j * _L, _L)] = dv[j]
                return carry
            lax.fori_loop(0, _R, body, jnp.int32(0))
        for ob in obs:
            init(ob)

        def start_in(c, b):
            return pltpu.async_copy(
                params_hbm.at[pl.ds(base_row + c * _R, _R), :],
                ins[b], sin[b])

        def start_out(c, b):
            return pltpu.async_copy(
                obs[b],
                out_hbm.at[pl.ds(base_row + c * _R, _R), :],
                sout[b])

        def compute(b):
            inb = ins[b]
            ob = obs[b]
            def body(r, carry):
                rvec = carry + r
                for k in range(F // _L):
                    x = inb[r, pl.ds(k * _L, _L)]
                    plsc.store_scatter(ob, [rvec, fi[k]], x)
                return carry
            lax.fori_loop(0, _R, body, zero16)

        in_cp = [None] * _NB
        out_cp = [None] * _NB
        for j in range(min(_NB, n_chunks)):
            in_cp[j] = start_in(j, j)
        for c in range(n_chunks):
            b = c % _NB
            in_cp[b].wait()
            if out_cp[b] is not None:
                out_cp[b].wait()
            compute(b)
            out_cp[b] = start_out(c, b)
            if c + _NB < n_chunks:
                in_cp[b] = start_in(c + _NB, b)
        for b in range(_NB):
            if out_cp[b] is not None:
                out_cp[b].wait()

    return kfn


@functools.lru_cache(maxsize=None)
def _build_tc_call(B: int, Bs: int, P: int, F: int):
    nblk = (B - Bs) // _TC_BM
    blk0 = Bs // _TC_BM
    assert (B - Bs) % _TC_BM == 0 and Bs % _TC_BM == 0

    def body(p_ref, fi_ref, d_ref, o_ref):
        fi = fi_ref[...]
        smat = (lax.broadcasted_iota(jnp.int32, (F, P), 1)
                == fi[:, None]).astype(jnp.float32)
        dflt = d_ref[...]
        dfix = dflt * (1.0 - jnp.max(smat, axis=0))
        o_ref[...] = lax.dot_general(
            p_ref[...], smat,
            (((1,), (0,)), ((), ())),
            precision=lax.Precision.HIGHEST,
            preferred_element_type=jnp.float32,
        ) + dfix[None, :]

    return pl.pallas_call(
        body,
        grid=(nblk,),
        in_specs=[
            pl.BlockSpec((_TC_BM, F), lambda i: (i + blk0, 0)),
            pl.BlockSpec((F,), lambda i: (0,)),
            pl.BlockSpec((P,), lambda i: (0,)),
        ],
        out_specs=pl.BlockSpec((_TC_BM, P), lambda i: (i + blk0, 0)),
        out_shape=jax.ShapeDtypeStruct((B, P), jnp.float32),
    )


def kernel(params, params_default, free_inds):
    B, F = params.shape
    P = params_default.shape[0]
    fi = free_inds.astype(jnp.int32)
    kfn = _build_sc_call(B, _B_SC, P, F)
    out_sc = kfn(params, fi, params_default)
    tfn = _build_tc_call(B, _B_SC, P, F)
    out_tc = tfn(params, fi, params_default)
    return lax.dynamic_update_slice(out_tc, out_sc, (0, 0))


# speedup vs baseline: 1.0354x; 1.0354x over previous
import functools

import jax
import jax.numpy as jnp
from jax import lax
from jax.experimental import pallas as pl
from jax.experimental.pallas import tpu as pltpu
from jax.experimental.pallas import tpu_sc as plsc

_NC = 2
_NS = 16
_NW = _NC * _NS
_R = 128
_NB = 3
_L = 16
_B_SC = 53248
_TC_BM = 2048


@functools.lru_cache(maxsize=None)
def _build_sc_call(B: int, Bs: int, P: int, F: int):
    rows_w = Bs // _NW
    n_chunks = rows_w // _R
    assert rows_w % _R == 0 and Bs % _NW == 0 and F % _L == 0 and P % _L == 0

    mesh = plsc.VectorSubcoreMesh(core_axis_name="c", subcore_axis_name="s")

    scratch = (
        [pltpu.VMEM((_R, F), jnp.float32) for _ in range(_NB)]
        + [pltpu.VMEM((_R, P), jnp.float32) for _ in range(_NB)]
        + [pltpu.VMEM((F,), jnp.int32),
           pltpu.VMEM((P,), jnp.float32)]
        + [pltpu.SemaphoreType.DMA for _ in range(2 * _NB + 1)]
    )

    @functools.partial(
        pl.kernel,
        out_type=jax.ShapeDtypeStruct((Bs, P), jnp.float32),
        mesh=mesh,
        compiler_params=pltpu.CompilerParams(
            use_tc_tiling_on_sc=True, needs_layout_passes=False),
        scratch_types=scratch,
    )
    def kfn(params_hbm, fi_hbm, dflt_hbm, out_hbm, *refs):
        ins = refs[:_NB]
        obs = refs[_NB:2 * _NB]
        fi_v, dflt_v = refs[2 * _NB:2 * _NB + 2]
        sin = refs[2 * _NB + 2:3 * _NB + 2]
        sout = refs[3 * _NB + 2:4 * _NB + 2]
        sx = refs[4 * _NB + 2]

        wid = lax.axis_index("s") * _NC + lax.axis_index("c")
        base_row = wid * rows_w

        pltpu.async_copy(fi_hbm, fi_v, sx).wait()
        pltpu.async_copy(dflt_hbm, dflt_v, sx).wait()

        fi = [fi_v[pl.ds(k * _L, _L)] for k in range(F // _L)]
        dv = [dflt_v[pl.ds(j * _L, _L)] for j in range(P // _L)]

        zero16 = jnp.zeros((_L,), jnp.int32)

        def init(ob):
            def body(r, carry):
                for j in range(P // _L):
                    ob[r, pl.ds(j * _L, _L)] = dv[j]
                return carry
            lax.fori_loop(0, _R, body, jnp.int32(0))
        for ob in obs:
            init(ob)

        def start_in(c, b):
            return pltpu.async_copy(
                params_hbm.at[pl.ds(base_row + c * _R, _R), :],
                ins[b], sin[b])

        def start_out(c, b):
            return pltpu.async_copy(
                obs[b],
                out_hbm.at[pl.ds(base_row + c * _R, _R), :],
                sout[b])

        def compute(b):
            inb = ins[b]
            ob = obs[b]
            def body(r, carry):
                rvec = carry + r
                for k in range(F // _L):
                    x = inb[r, pl.ds(k * _L, _L)]
                    plsc.store_scatter(ob, [rvec, fi[k]], x)
                return carry
            lax.fori_loop(0, _R, body, zero16)

        in_cp = [None] * _NB
        out_cp = [None] * _NB
        for j in range(min(_NB, n_chunks)):
            in_cp[j] = start_in(j, j)
        for c in range(n_chunks):
            b = c % _NB
            in_cp[b].wait()
            if out_cp[b] is not None:
                out_cp[b].wait()
            compute(b)
            out_cp[b] = start_out(c, b)
            if c + _NB < n_chunks:
                in_cp[b] = start_in(c + _NB, b)
        for b in range(_NB):
            if out_cp[b] is not None:
                out_cp[b].wait()

    return kfn


@functools.lru_cache(maxsize=None)
def _build_tc_call(B: int, Bs: int, P: int, F: int):
    nblk = (B - Bs) // _TC_BM
    blk0 = Bs // _TC_BM
    assert (B - Bs) % _TC_BM == 0 and Bs % _TC_BM == 0

    def body(p_ref, fi_ref, d_ref, o_ref):
        fi = fi_ref[...]
        smat = (lax.broadcasted_iota(jnp.int32, (F, P), 1)
                == fi[:, None]).astype(jnp.float32)
        dflt = d_ref[...]
        dfix = dflt * (1.0 - jnp.max(smat, axis=0))
        o_ref[...] = lax.dot_general(
            p_ref[...], smat,
            (((1,), (0,)), ((), ())),
            precision=lax.Precision.HIGHEST,
            preferred_element_type=jnp.float32,
        ) + dfix[None, :]

    return pl.pallas_call(
        body,
        grid=(nblk,),
        in_specs=[
            pl.BlockSpec((_TC_BM, F), lambda i: (i + blk0, 0)),
            pl.BlockSpec((F,), lambda i: (0,)),
            pl.BlockSpec((P,), lambda i: (0,)),
        ],
        out_specs=pl.BlockSpec((_TC_BM, P), lambda i: (i + blk0, 0)),
        out_shape=jax.ShapeDtypeStruct((B, P), jnp.float32),
    )


def kernel(params, params_default, free_inds):
    B, F = params.shape
    P = params_default.shape[0]
    fi = free_inds.astype(jnp.int32)
    kfn = _build_sc_call(B, _B_SC, P, F)
    out_sc = kfn(params, fi, params_default)
    tfn = _build_tc_call(B, _B_SC, P, F)
    out_tc = tfn(params, fi, params_default)
    return lax.dynamic_update_slice(out_tc, out_sc, (0, 0))


# speedup vs baseline: 1.0485x; 1.0126x over previous
import functools

import jax
import jax.numpy as jnp
from jax import lax
from jax.experimental import pallas as pl
from jax.experimental.pallas import tpu as pltpu
from jax.experimental.pallas import tpu_sc as plsc

_NC = 2
_NS = 16
_NW = _NC * _NS
_R = 128
_NB = 3
_L = 16
_B_SC = 57344
_TC_BM = 2048


@functools.lru_cache(maxsize=None)
def _build_sc_call(B: int, Bs: int, P: int, F: int):
    rows_w = Bs // _NW
    n_chunks = rows_w // _R
    assert rows_w % _R == 0 and Bs % _NW == 0 and F % _L == 0 and P % _L == 0

    mesh = plsc.VectorSubcoreMesh(core_axis_name="c", subcore_axis_name="s")

    scratch = (
        [pltpu.VMEM((_R, F), jnp.float32) for _ in range(_NB)]
        + [pltpu.VMEM((_R, P), jnp.float32) for _ in range(_NB)]
        + [pltpu.VMEM((F,), jnp.int32),
           pltpu.VMEM((P,), jnp.float32)]
        + [pltpu.SemaphoreType.DMA for _ in range(2 * _NB + 1)]
    )

    @functools.partial(
        pl.kernel,
        out_type=jax.ShapeDtypeStruct((Bs, P), jnp.float32),
        mesh=mesh,
        compiler_params=pltpu.CompilerParams(
            use_tc_tiling_on_sc=True, needs_layout_passes=False),
        scratch_types=scratch,
    )
    def kfn(params_hbm, fi_hbm, dflt_hbm, out_hbm, *refs):
        ins = refs[:_NB]
        obs = refs[_NB:2 * _NB]
        fi_v, dflt_v = refs[2 * _NB:2 * _NB + 2]
        sin = refs[2 * _NB + 2:3 * _NB + 2]
        sout = refs[3 * _NB + 2:4 * _NB + 2]
        sx = refs[4 * _NB + 2]

        wid = lax.axis_index("s") * _NC + lax.axis_index("c")
        base_row = wid * rows_w

        pltpu.async_copy(fi_hbm, fi_v, sx).wait()
        pltpu.async_copy(dflt_hbm, dflt_v, sx).wait()

        fi = [fi_v[pl.ds(k * _L, _L)] for k in range(F // _L)]
        dv = [dflt_v[pl.ds(j * _L, _L)] for j in range(P // _L)]

        zero16 = jnp.zeros((_L,), jnp.int32)

        def init(ob):
            def body(r, carry):
                for j in range(P // _L):
                    ob[r, pl.ds(j * _L, _L)] = dv[j]
                return carry
            lax.fori_loop(0, _R, body, jnp.int32(0))
        for ob in obs:
            init(ob)

        def start_in(c, b):
            return pltpu.async_copy(
                params_hbm.at[pl.ds(base_row + c * _R, _R), :],
                ins[b], sin[b])

        def start_out(c, b):
            return pltpu.async_copy(
                obs[b],
                out_hbm.at[pl.ds(base_row + c * _R, _R), :],
                sout[b])

        def compute(b):
            inb = ins[b]
            ob = obs[b]
            def body(r, carry):
                rvec = carry + r
                for k in range(F // _L):
                    x = inb[r, pl.ds(k * _L, _L)]
                    plsc.store_scatter(ob, [rvec, fi[k]], x)
                return carry
            lax.fori_loop(0, _R, body, zero16)

        in_cp = [None] * _NB
        out_cp = [None] * _NB
        for j in range(min(_NB, n_chunks)):
            in_cp[j] = start_in(j, j)
        for c in range(n_chunks):
            b = c % _NB
            in_cp[b].wait()
            if out_cp[b] is not None:
                out_cp[b].wait()
            compute(b)
            out_cp[b] = start_out(c, b)
            if c + _NB < n_chunks:
                in_cp[b] = start_in(c + _NB, b)
        for b in range(_NB):
            if out_cp[b] is not None:
                out_cp[b].wait()

    return kfn


@functools.lru_cache(maxsize=None)
def _build_tc_call(B: int, Bs: int, P: int, F: int):
    nblk = (B - Bs) // _TC_BM
    blk0 = Bs // _TC_BM
    assert (B - Bs) % _TC_BM == 0 and Bs % _TC_BM == 0

    def body(p_ref, fi_ref, d_ref, o_ref):
        fi = fi_ref[...]
        smat = (lax.broadcasted_iota(jnp.int32, (F, P), 1)
                == fi[:, None]).astype(jnp.float32)
        dflt = d_ref[...]
        dfix = dflt * (1.0 - jnp.max(smat, axis=0))
        o_ref[...] = lax.dot_general(
            p_ref[...], smat,
            (((1,), (0,)), ((), ())),
            precision=lax.Precision.HIGHEST,
            preferred_element_type=jnp.float32,
        ) + dfix[None, :]

    return pl.pallas_call(
        body,
        grid=(nblk,),
        in_specs=[
            pl.BlockSpec((_TC_BM, F), lambda i: (i + blk0, 0)),
            pl.BlockSpec((F,), lambda i: (0,)),
            pl.BlockSpec((P,), lambda i: (0,)),
        ],
        out_specs=pl.BlockSpec((_TC_BM, P), lambda i: (i + blk0, 0)),
        out_shape=jax.ShapeDtypeStruct((B, P), jnp.float32),
    )


def kernel(params, params_default, free_inds):
    B, F = params.shape
    P = params_default.shape[0]
    fi = free_inds.astype(jnp.int32)
    kfn = _build_sc_call(B, _B_SC, P, F)
    out_sc = kfn(params, fi, params_default)
    tfn = _build_tc_call(B, _B_SC, P, F)
    out_tc = tfn(params, fi, params_default)
    return lax.dynamic_update_slice(out_tc, out_sc, (0, 0))


# speedup vs baseline: 1.0538x; 1.0051x over previous
import functools

import jax
import jax.numpy as jnp
from jax import lax
from jax.experimental import pallas as pl
from jax.experimental.pallas import tpu as pltpu
from jax.experimental.pallas import tpu_sc as plsc

_NC = 2
_NS = 16
_NW = _NC * _NS
_R = 128
_NB = 3
_L = 16
_B_SC = 61440
_TC_BM = 2048


@functools.lru_cache(maxsize=None)
def _build_sc_call(B: int, Bs: int, P: int, F: int):
    rows_w = Bs // _NW
    n_chunks = rows_w // _R
    assert rows_w % _R == 0 and Bs % _NW == 0 and F % _L == 0 and P % _L == 0

    mesh = plsc.VectorSubcoreMesh(core_axis_name="c", subcore_axis_name="s")

    scratch = (
        [pltpu.VMEM((_R, F), jnp.float32) for _ in range(_NB)]
        + [pltpu.VMEM((_R, P), jnp.float32) for _ in range(_NB)]
        + [pltpu.VMEM((F,), jnp.int32),
           pltpu.VMEM((P,), jnp.float32)]
        + [pltpu.SemaphoreType.DMA for _ in range(2 * _NB + 1)]
    )

    @functools.partial(
        pl.kernel,
        out_type=jax.ShapeDtypeStruct((Bs, P), jnp.float32),
        mesh=mesh,
        compiler_params=pltpu.CompilerParams(
            use_tc_tiling_on_sc=True, needs_layout_passes=False),
        scratch_types=scratch,
    )
    def kfn(params_hbm, fi_hbm, dflt_hbm, out_hbm, *refs):
        ins = refs[:_NB]
        obs = refs[_NB:2 * _NB]
        fi_v, dflt_v = refs[2 * _NB:2 * _NB + 2]
        sin = refs[2 * _NB + 2:3 * _NB + 2]
        sout = refs[3 * _NB + 2:4 * _NB + 2]
        sx = refs[4 * _NB + 2]

        wid = lax.axis_index("s") * _NC + lax.axis_index("c")
        base_row = wid * rows_w

        pltpu.async_copy(fi_hbm, fi_v, sx).wait()
        pltpu.async_copy(dflt_hbm, dflt_v, sx).wait()

        fi = [fi_v[pl.ds(k * _L, _L)] for k in range(F // _L)]
        dv = [dflt_v[pl.ds(j * _L, _L)] for j in range(P // _L)]

        zero16 = jnp.zeros((_L,), jnp.int32)

        def init(ob):
            def body(r, carry):
                for j in range(P // _L):
                    ob[r, pl.ds(j * _L, _L)] = dv[j]
                return carry
            lax.fori_loop(0, _R, body, jnp.int32(0))
        for ob in obs:
            init(ob)

        def start_in(c, b):
            return pltpu.async_copy(
                params_hbm.at[pl.ds(base_row + c * _R, _R), :],
                ins[b], sin[b])

        def start_out(c, b):
            return pltpu.async_copy(
                obs[b],
                out_hbm.at[pl.ds(base_row + c * _R, _R), :],
                sout[b])

        def compute(b):
            inb = ins[b]
            ob = obs[b]
            def body(r, carry):
                rvec = carry + r
                for k in range(F // _L):
                    x = inb[r, pl.ds(k * _L, _L)]
                    plsc.store_scatter(ob, [rvec, fi[k]], x)
                return carry
            lax.fori_loop(0, _R, body, zero16)

        in_cp = [None] * _NB
        out_cp = [None] * _NB
        for j in range(min(_NB, n_chunks)):
            in_cp[j] = start_in(j, j)
        for c in range(n_chunks):
            b = c % _NB
            in_cp[b].wait()
            if out_cp[b] is not None:
                out_cp[b].wait()
            compute(b)
            out_cp[b] = start_out(c, b)
            if c + _NB < n_chunks:
                in_cp[b] = start_in(c + _NB, b)
        for b in range(_NB):
            if out_cp[b] is not None:
                out_cp[b].wait()

    return kfn


@functools.lru_cache(maxsize=None)
def _build_tc_call(B: int, Bs: int, P: int, F: int):
    nblk = (B - Bs) // _TC_BM
    blk0 = Bs // _TC_BM
    assert (B - Bs) % _TC_BM == 0 and Bs % _TC_BM == 0

    def body(p_ref, fi_ref, d_ref, o_ref):
        fi = fi_ref[...]
        smat = (lax.broadcasted_iota(jnp.int32, (F, P), 1)
                == fi[:, None]).astype(jnp.float32)
        dflt = d_ref[...]
        dfix = dflt * (1.0 - jnp.max(smat, axis=0))
        o_ref[...] = lax.dot_general(
            p_ref[...], smat,
            (((1,), (0,)), ((), ())),
            precision=lax.Precision.HIGHEST,
            preferred_element_type=jnp.float32,
        ) + dfix[None, :]

    return pl.pallas_call(
        body,
        grid=(nblk,),
        in_specs=[
            pl.BlockSpec((_TC_BM, F), lambda i: (i + blk0, 0)),
            pl.BlockSpec((F,), lambda i: (0,)),
            pl.BlockSpec((P,), lambda i: (0,)),
        ],
        out_specs=pl.BlockSpec((_TC_BM, P), lambda i: (i + blk0, 0)),
        out_shape=jax.ShapeDtypeStruct((B, P), jnp.float32),
    )


def kernel(params, params_default, free_inds):
    B, F = params.shape
    P = params_default.shape[0]
    fi = free_inds.astype(jnp.int32)
    kfn = _build_sc_call(B, _B_SC, P, F)
    out_sc = kfn(params, fi, params_default)
    tfn = _build_tc_call(B, _B_SC, P, F)
    out_tc = tfn(params, fi, params_default)
    return lax.dynamic_update_slice(out_tc, out_sc, (0, 0))
